# baseline (speedup 1.0000x reference)
import jax
import jax.numpy as jnp
from jax.experimental import pallas as pl
from jax.experimental.pallas import tpu as pltpu

_ROWS = 16384
_COLS = 128
_PANELS = _ROWS // 128
_PB = 16
_STEPS = _PANELS // _PB


def _gather_cols_kernel(x_hbm, o_ref, buf, sem):
    step = pl.program_id(0)

    def start(i, slot):
        pltpu.make_async_copy(
            x_hbm.at[pl.ds(i * _PB * 128, _PB * 128), :], buf.at[slot], sem.at[slot]
        ).start()

    @pl.when(step == 0)
    def _():
        start(0, 0)

    @pl.when(step + 1 < _STEPS)
    def _():
        start(step + 1, (step + 1) % 2)

    slot = step % 2
    pltpu.make_async_copy(
        x_hbm.at[pl.ds(step * _PB * 128, _PB * 128), :], buf.at[slot], sem.at[slot]
    ).wait()

    y = buf[slot].reshape(_PB, 128, _COLS)
    z = jnp.concatenate(
        [y[:, :, 0:1], y[:, :, 1:2], y[:, :, 4:5], y[:, :, 4:5]], axis=2
    )
    o_ref[...] = jnp.transpose(z, (0, 2, 1))


def kernel(x):
    x = pltpu.with_memory_space_constraint(x, pltpu.MemorySpace.HBM)
    t = pl.pallas_call(
        _gather_cols_kernel,
        grid=(_STEPS,),
        in_specs=[pl.BlockSpec(memory_space=pl.ANY)],
        out_specs=pl.BlockSpec((_PB, 4, 128), lambda i: (i, 0, 0)),
        out_shape=jax.ShapeDtypeStruct((_PANELS, 4, 128), jnp.float32),
        scratch_shapes=[
            pltpu.VMEM((2, _PB * 128, _COLS), jnp.float32),
            pltpu.SemaphoreType.DMA((2,)),
        ],
    )(x)
    return jnp.transpose(t, (0, 2, 1)).reshape(_ROWS, 4)


# speedup vs baseline: 1.8560x; 1.8560x over previous
import jax
import jax.numpy as jnp
from jax import lax
from jax.experimental import pallas as pl
from jax.experimental.pallas import tpu as pltpu

_ROWS = 16384
_COLS = 128
_PANELS = _ROWS // 128
_PB = 32
_STEPS = _PANELS // _PB
_SRC = (0, 1, 4, 4)


def _gather_cols_kernel(x_hbm, o_ref, buf, sem):
    step = pl.program_id(0)

    def start(i, slot):
        pltpu.make_async_copy(
            x_hbm.at[pl.ds(i * _PB * 128, _PB * 128), :], buf.at[slot], sem.at[slot]
        ).start()

    @pl.when(step == 0)
    def _():
        start(0, 0)

    @pl.when(step + 1 < _STEPS)
    def _():
        start(step + 1, (step + 1) % 2)

    slot = step % 2
    pltpu.make_async_copy(
        x_hbm.at[pl.ds(step * _PB * 128, _PB * 128), :], buf.at[slot], sem.at[slot]
    ).wait()

    k_idx = lax.broadcasted_iota(jnp.int32, (4, _COLS), 1)
    c_idx = lax.broadcasted_iota(jnp.int32, (4, _COLS), 0)
    src = jnp.where(c_idx >= 2, 4, c_idx)
    sel = jnp.where(k_idx == src, 1.0, 0.0)
    ot = lax.dot_general(
        sel,
        buf[slot],
        (((1,), (1,)), ((), ())),
        preferred_element_type=jnp.float32,
    )
    for p in range(_PB):
        o_ref[p] = ot[:, p * 128 : (p + 1) * 128]


def kernel(x):
    x = pltpu.with_memory_space_constraint(x, pltpu.MemorySpace.HBM)
    t = pl.pallas_call(
        _gather_cols_kernel,
        grid=(_STEPS,),
        in_specs=[pl.BlockSpec(memory_space=pl.ANY)],
        out_specs=pl.BlockSpec((_PB, 4, 128), lambda i: (i, 0, 0)),
        out_shape=jax.ShapeDtypeStruct((_PANELS, 4, 128), jnp.float32),
        scratch_shapes=[
            pltpu.VMEM((2, _PB * 128, _COLS), jnp.float32),
            pltpu.SemaphoreType.DMA((2,)),
        ],
    )(x)
    return jnp.transpose(t, (0, 2, 1)).reshape(_ROWS, 4)
